# speedup vs baseline: 7.9181x; 7.9181x over previous
import jax
import jax.numpy as jnp
from jax import lax
from jax.experimental import pallas as pl
from jax.experimental.pallas import tpu as pltpu
from jax.experimental.pallas import tpu_sc as plsc

D = 128
N = 1024 * 200
NW = 32
PER_W = N // NW
CH = 128
NCH = PER_W // CH
NBUF = 2


def _embed_body(tok_hbm, tab_hbm, out_hbm, idx_v, rows_v, sem0, sem1):
    sems = (sem0, sem1)
    wid = lax.axis_index("s") * 2 + lax.axis_index("c")
    base = wid * PER_W

    pltpu.sync_copy(tok_hbm.at[wid], idx_v)

    for b in range(NBUF):
        pltpu.async_copy(tab_hbm.at[idx_v.at[b]], rows_v.at[b], sems[b])

    def body(i, carry):
        for bb in range(NBUF):
            j = i * NBUF + bb
            pltpu.make_async_copy(
                tab_hbm.at[idx_v.at[0]], rows_v.at[bb], sems[bb]
            ).wait()
            pltpu.sync_copy(
                rows_v.at[bb], out_hbm.at[pl.ds(base + j * CH, CH)]
            )
            nxt = j + NBUF

            @pl.when(nxt < NCH)
            def _():
                pltpu.async_copy(
                    tab_hbm.at[idx_v.at[nxt]], rows_v.at[bb], sems[bb]
                )
        return carry

    lax.fori_loop(0, NCH // NBUF, body, 0)


def kernel(tokens, table):
    tok3 = tokens.reshape(NW, NCH, CH)
    mesh = plsc.VectorSubcoreMesh(core_axis_name="c", subcore_axis_name="s")
    out = pl.kernel(
        _embed_body,
        out_type=jax.ShapeDtypeStruct((N, D), jnp.float32),
        mesh=mesh,
        scratch_types=[
            pltpu.VMEM((NCH, CH), jnp.int32),
            pltpu.VMEM((NBUF, CH, D), jnp.float32),
            pltpu.SemaphoreType.DMA,
            pltpu.SemaphoreType.DMA,
        ],
    )(tok3, table)
    return out.reshape(tokens.shape[0], tokens.shape[1], D)


# speedup vs baseline: 8.0575x; 1.0176x over previous
import jax
import jax.numpy as jnp
from jax import lax
from jax.experimental import pallas as pl
from jax.experimental.pallas import tpu as pltpu
from jax.experimental.pallas import tpu_sc as plsc

D = 128
N = 1024 * 200
NW = 32
PER_W = N // NW
CH = 128
NCH = PER_W // CH
NBUF = 5


def _embed_body(tok_hbm, tab_hbm, out_hbm, idx_v, rows_v, *sems):
    wid = lax.axis_index("s") * 2 + lax.axis_index("c")
    base = wid * PER_W

    pltpu.sync_copy(tok_hbm.at[wid], idx_v)

    for b in range(NBUF):
        pltpu.async_copy(tab_hbm.at[idx_v.at[b]], rows_v.at[b], sems[b])

    def body(i, carry):
        for bb in range(NBUF):
            j = i * NBUF + bb
            pltpu.make_async_copy(
                tab_hbm.at[idx_v.at[0]], rows_v.at[bb], sems[bb]
            ).wait()
            pltpu.sync_copy(
                rows_v.at[bb], out_hbm.at[pl.ds(base + j * CH, CH)]
            )
            nxt = j + NBUF

            @pl.when(nxt < NCH)
            def _():
                pltpu.async_copy(
                    tab_hbm.at[idx_v.at[nxt]], rows_v.at[bb], sems[bb]
                )
        return carry

    lax.fori_loop(0, NCH // NBUF, body, 0)


def kernel(tokens, table):
    tok3 = tokens.reshape(NW, NCH, CH)
    mesh = plsc.VectorSubcoreMesh(core_axis_name="c", subcore_axis_name="s")
    out = pl.kernel(
        _embed_body,
        out_type=jax.ShapeDtypeStruct((N, D), jnp.float32),
        mesh=mesh,
        scratch_types=[
            pltpu.VMEM((NCH, CH), jnp.int32),
            pltpu.VMEM((NBUF, CH, D), jnp.float32),
        ] + [pltpu.SemaphoreType.DMA] * NBUF,
    )(tok3, table)
    return out.reshape(tokens.shape[0], tokens.shape[1], D)


# speedup vs baseline: 8.0608x; 1.0004x over previous
import jax
import jax.numpy as jnp
from jax import lax
from jax.experimental import pallas as pl
from jax.experimental.pallas import tpu as pltpu
from jax.experimental.pallas import tpu_sc as plsc

D = 128
N = 1024 * 200
NW = 32
PER_W = N // NW
CH = 128
NCH = PER_W // CH
NBUF = 5
G = 3


def _embed_body(tok_hbm, tab_hbm, out_hbm, idx_v, rows_v, *sems):
    gsems = sems[:NBUF]
    ssems = sems[NBUF:]
    wid = lax.axis_index("s") * 2 + lax.axis_index("c")
    base = wid * PER_W

    pltpu.sync_copy(tok_hbm.at[wid], idx_v)

    for b in range(G):
        pltpu.async_copy(tab_hbm.at[idx_v.at[b]], rows_v.at[b], gsems[b])

    def wait_gather(bb):
        pltpu.make_async_copy(
            tab_hbm.at[idx_v.at[0]], rows_v.at[bb], gsems[bb]
        ).wait()

    def wait_store(bb):
        pltpu.make_async_copy(
            rows_v.at[bb], out_hbm.at[pl.ds(0, CH)], ssems[bb]
        ).wait()

    def body(i, carry):
        for bb in range(NBUF):
            j = i * NBUF + bb
            wait_gather(bb)
            pltpu.async_copy(
                rows_v.at[bb], out_hbm.at[pl.ds(base + j * CH, CH)], ssems[bb]
            )
            nb = (bb + G) % NBUF
            nxt = j + G

            @pl.when(nxt < NCH)
            def _():
                @pl.when(nxt - NBUF >= 0)
                def _():
                    wait_store(nb)

                pltpu.async_copy(
                    tab_hbm.at[idx_v.at[nxt]], rows_v.at[nb], gsems[nb]
                )
        return carry

    lax.fori_loop(0, NCH // NBUF, body, 0)

    for bb in range(NBUF):
        wait_store(bb)


def kernel(tokens, table):
    tok3 = tokens.reshape(NW, NCH, CH)
    mesh = plsc.VectorSubcoreMesh(core_axis_name="c", subcore_axis_name="s")
    out = pl.kernel(
        _embed_body,
        out_type=jax.ShapeDtypeStruct((N, D), jnp.float32),
        mesh=mesh,
        scratch_types=[
            pltpu.VMEM((NCH, CH), jnp.int32),
            pltpu.VMEM((NBUF, CH, D), jnp.float32),
        ] + [pltpu.SemaphoreType.DMA] * (2 * NBUF),
    )(tok3, table)
    return out.reshape(tokens.shape[0], tokens.shape[1], D)
